# baseline (speedup 1.0000x reference)
import functools

import jax
import jax.numpy as jnp
from jax import lax
from jax.experimental import pallas as pl
from jax.experimental.pallas import tpu as pltpu
from jax.experimental.pallas import tpu_sc as plsc

_N_E = 8192
_E_DIM = 32
_BETA = 0.25

_TOK_BLOCK = 4096
_K_CHUNK = 2048
_N_KC = _N_E // _K_CHUNK
_N_TB = 8192 // _TOK_BLOCK


def _argmin_body(z_ref, zsq_ref, embt_ref, esq_ref, idx_ref, dmin_ref):
    zb = z_ref[...].astype(jnp.bfloat16).astype(jnp.float32)
    zsq = zsq_ref[...]

    def chunk_min(k):
        embt2 = embt_ref[:, pl.ds(k * _K_CHUNK, _K_CHUNK)]
        esq = esq_ref[0, pl.ds(k * _K_CHUNK, _K_CHUNK)]
        mm2 = lax.dot_general(zb, embt2, (((1,), (0,)), ((), ())),
                              preferred_element_type=jnp.float32)
        lane = lax.broadcasted_iota(jnp.int32, (_TOK_BLOCK, 128), 1)

        def d_tile(t):
            return (zsq + esq[None, t * 128:(t + 1) * 128]) \
                - mm2[:, t * 128:(t + 1) * 128]

        run_val = d_tile(0)
        run_idx = lane
        for t in range(1, _K_CHUNK // 128):
            dv = d_tile(t)
            upd = dv < run_val
            run_val = jnp.where(upd, dv, run_val)
            run_idx = jnp.where(upd, lane + t * 128, run_idx)
        cmin = jnp.min(run_val, axis=1)
        cidx = jnp.min(jnp.where(run_val == cmin[:, None], run_idx, _N_E),
                       axis=1)
        return cmin, cidx + k * _K_CHUNK

    c0_min, c0_idx = chunk_min(0)
    bv16 = c0_min.astype(jnp.bfloat16).astype(jnp.float32)
    best_idx, best_w = c0_idx, c0_min
    for k in range(1, _N_KC):
        cmin, cidx = chunk_min(k)
        upd = cmin < bv16
        bv16 = jnp.where(upd, cmin.astype(jnp.bfloat16).astype(jnp.float32),
                         bv16)
        best_idx = jnp.where(upd, cidx, best_idx)
        best_w = jnp.where(upd, cmin, best_w)
    idx_ref[...] = best_idx.reshape(1, 1, _TOK_BLOCK)
    dmin_ref[...] = best_w.reshape(1, 1, _TOK_BLOCK)


def _distance_argmin(z_flat, zsq, embt, esq):
    return pl.pallas_call(
        _argmin_body,
        grid=(_N_TB,),
        in_specs=[
            pl.BlockSpec((_TOK_BLOCK, _E_DIM), lambda i: (i, 0)),
            pl.BlockSpec((_TOK_BLOCK, 1), lambda i: (i, 0)),
            pl.BlockSpec((_E_DIM, _N_E), lambda i: (0, 0)),
            pl.BlockSpec((1, _N_E), lambda i: (0, 0)),
        ],
        out_specs=[
            pl.BlockSpec((1, 1, _TOK_BLOCK), lambda i: (i, 0, 0)),
            pl.BlockSpec((1, 1, _TOK_BLOCK), lambda i: (i, 0, 0)),
        ],
        out_shape=[
            jax.ShapeDtypeStruct((_N_TB, 1, _TOK_BLOCK), jnp.int32),
            jax.ShapeDtypeStruct((_N_TB, 1, _TOK_BLOCK), jnp.float32),
        ],
    )(z_flat, zsq, embt, esq)


try:
    _SC_INFO = plsc.get_sparse_core_info()
    _NC = _SC_INFO.num_cores
    _NS = _SC_INFO.num_subcores
except Exception:
    _NC, _NS = 2, 16
_NW = _NC * _NS
_BPW = 8192 // _NW
_GCH = 128
_NGC = _BPW // _GCH


_ROW_PAD = 128


def _gather_st_body(table_hbm, idx_hbm, z_hbm, out_hbm, idx_v, rows_v, z_v,
                    out_v, sem):
    wid = lax.axis_index("s") * _NC + lax.axis_index("c")
    base = wid * _BPW
    for j in range(_NGC):
        pltpu.sync_copy(idx_hbm.at[pl.ds(base + j * _GCH, _GCH)], idx_v.at[j])
    copies = [pltpu.async_copy(table_hbm.at[idx_v.at[j]],
                               rows_v.at[pl.ds(j * _GCH, _GCH)], sem)
              for j in range(_NGC)]
    pltpu.sync_copy(z_hbm.at[pl.ds(base, _BPW)], z_v)
    for c in copies:
        c.wait()

    def body(r, carry):
        for c in range(_E_DIM // 16):
            zz = z_v[r, pl.ds(c * 16, 16)]
            q = rows_v[r, pl.ds(c * 16, 16)]
            out_v[r, pl.ds(c * 16, 16)] = zz + (q - zz)
        return carry

    lax.fori_loop(0, _BPW, body, 0)
    pltpu.sync_copy(out_v, out_hbm.at[pl.ds(base, _BPW)])


def _gather_straight_through(table_padded, idx_flat, z_flat):
    mesh = plsc.VectorSubcoreMesh(core_axis_name="c", subcore_axis_name="s")
    fn = functools.partial(
        pl.kernel,
        mesh=mesh,
        out_type=jax.ShapeDtypeStruct((8192, _E_DIM), jnp.float32),
        scratch_types=[
            pltpu.VMEM((_NGC, _GCH), jnp.int32),
            pltpu.VMEM((_BPW, _ROW_PAD), jnp.float32),
            pltpu.VMEM((_BPW, _E_DIM), jnp.float32),
            pltpu.VMEM((_BPW, _E_DIM), jnp.float32),
            pltpu.SemaphoreType.DMA,
        ],
    )(_gather_st_body)
    return fn(table_padded, idx_flat, z_flat)


def kernel(z, embedding):
    z_flat = z.reshape(-1, _E_DIM)
    zsq = jnp.sum(z_flat ** 2, axis=1, keepdims=True)
    esq = jnp.sum(embedding ** 2, axis=1)
    embt2 = embedding.T * 2.0

    idx_blocks, dmin_blocks = _distance_argmin(z_flat, zsq, embt2,
                                               esq.reshape(1, _N_E))
    encoding_indices = idx_blocks.reshape(z.shape[:-1])
    idx_flat = idx_blocks.reshape(-1)

    table_padded = jnp.pad(embedding, ((0, 0), (0, _ROW_PAD - _E_DIM)))
    z_q_out = _gather_straight_through(table_padded, idx_flat, z_flat)
    z_q_out = z_q_out.reshape(z.shape)

    m = jnp.sum(dmin_blocks) / (8192.0 * _E_DIM)
    loss = _BETA * m + m
    return (z_q_out, loss, encoding_indices)


# speedup vs baseline: 1.3068x; 1.3068x over previous
import functools

import jax
import jax.numpy as jnp
from jax import lax
from jax.experimental import pallas as pl
from jax.experimental.pallas import tpu as pltpu
from jax.experimental.pallas import tpu_sc as plsc

_N_E = 8192
_E_DIM = 32
_BETA = 0.25

_TOK_BLOCK = 2048
_K_CHUNK = 2048
_N_KC = _N_E // _K_CHUNK
_N_TB = 8192 // _TOK_BLOCK


def _argmin_body(z_ref, zsq_ref, embt_ref, esq_ref, idx_ref, dmin_ref):
    zb = z_ref[...].astype(jnp.bfloat16).astype(jnp.float32)
    zsq = zsq_ref[...]

    def chunk_min(k):
        embt2 = embt_ref[:, pl.ds(k * _K_CHUNK, _K_CHUNK)]
        esq = esq_ref[0, pl.ds(k * _K_CHUNK, _K_CHUNK)]
        mm2 = lax.dot_general(zb, embt2, (((1,), (0,)), ((), ())),
                              preferred_element_type=jnp.float32)
        lane = lax.broadcasted_iota(jnp.int32, (_TOK_BLOCK, 128), 1)

        def d_tile(t):
            return (zsq + esq[None, t * 128:(t + 1) * 128]) \
                - mm2[:, t * 128:(t + 1) * 128]

        run_val = d_tile(0)
        run_idx = lane
        for t in range(1, _K_CHUNK // 128):
            dv = d_tile(t)
            upd = dv < run_val
            run_val = jnp.where(upd, dv, run_val)
            run_idx = jnp.where(upd, lane + t * 128, run_idx)
        cmin = jnp.min(run_val, axis=1)
        cidx = jnp.min(jnp.where(run_val == cmin[:, None], run_idx, _N_E),
                       axis=1)
        return cmin, cidx + k * _K_CHUNK

    c0_min, c0_idx = chunk_min(0)
    bv16 = c0_min.astype(jnp.bfloat16).astype(jnp.float32)
    best_idx, best_w = c0_idx, c0_min
    for k in range(1, _N_KC):
        cmin, cidx = chunk_min(k)
        upd = cmin < bv16
        bv16 = jnp.where(upd, cmin.astype(jnp.bfloat16).astype(jnp.float32),
                         bv16)
        best_idx = jnp.where(upd, cidx, best_idx)
        best_w = jnp.where(upd, cmin, best_w)
    idx_ref[...] = best_idx.reshape(1, 1, _TOK_BLOCK)
    dmin_ref[...] = best_w.reshape(1, 1, _TOK_BLOCK)


def _distance_argmin(z_flat, zsq, embt, esq):
    return pl.pallas_call(
        _argmin_body,
        grid=(_N_TB,),
        in_specs=[
            pl.BlockSpec((_TOK_BLOCK, _E_DIM), lambda i: (i, 0)),
            pl.BlockSpec((_TOK_BLOCK, 1), lambda i: (i, 0)),
            pl.BlockSpec((_E_DIM, _N_E), lambda i: (0, 0)),
            pl.BlockSpec((1, _N_E), lambda i: (0, 0)),
        ],
        out_specs=[
            pl.BlockSpec((1, 1, _TOK_BLOCK), lambda i: (i, 0, 0)),
            pl.BlockSpec((1, 1, _TOK_BLOCK), lambda i: (i, 0, 0)),
        ],
        out_shape=[
            jax.ShapeDtypeStruct((_N_TB, 1, _TOK_BLOCK), jnp.int32),
            jax.ShapeDtypeStruct((_N_TB, 1, _TOK_BLOCK), jnp.float32),
        ],
    )(z_flat, zsq, embt, esq)


try:
    _SC_INFO = plsc.get_sparse_core_info()
    _NC = _SC_INFO.num_cores
    _NS = _SC_INFO.num_subcores
except Exception:
    _NC, _NS = 2, 16
_NW = _NC * _NS
_BPW = 8192 // _NW
_GCH = 128
_NGC = _BPW // _GCH


_ROW_PAD = 128


def _gather_st_body(table_hbm, idx_hbm, z_hbm, out_hbm, idx_v, rows_v, z_v,
                    out_v, sem):
    wid = lax.axis_index("s") * _NC + lax.axis_index("c")
    base = wid * _BPW
    for j in range(_NGC):
        pltpu.sync_copy(idx_hbm.at[pl.ds(base + j * _GCH, _GCH)], idx_v.at[j])
    copies = [pltpu.async_copy(table_hbm.at[idx_v.at[j]],
                               rows_v.at[pl.ds(j * _GCH, _GCH)], sem)
              for j in range(_NGC)]
    pltpu.sync_copy(z_hbm.at[pl.ds(base, _BPW)], z_v)
    for c in copies:
        c.wait()

    def body(r, carry):
        for c in range(_E_DIM // 16):
            zz = z_v[r, pl.ds(c * 16, 16)]
            q = rows_v[r, pl.ds(c * 16, 16)]
            out_v[r, pl.ds(c * 16, 16)] = zz + (q - zz)
        return carry

    lax.fori_loop(0, _BPW, body, 0)
    pltpu.sync_copy(out_v, out_hbm.at[pl.ds(base, _BPW)])


def _gather_straight_through(table_padded, idx_flat, z_flat):
    mesh = plsc.VectorSubcoreMesh(core_axis_name="c", subcore_axis_name="s")
    fn = functools.partial(
        pl.kernel,
        mesh=mesh,
        out_type=jax.ShapeDtypeStruct((8192, _E_DIM), jnp.float32),
        scratch_types=[
            pltpu.VMEM((_NGC, _GCH), jnp.int32),
            pltpu.VMEM((_BPW, _ROW_PAD), jnp.float32),
            pltpu.VMEM((_BPW, _E_DIM), jnp.float32),
            pltpu.VMEM((_BPW, _E_DIM), jnp.float32),
            pltpu.SemaphoreType.DMA,
        ],
    )(_gather_st_body)
    return fn(table_padded, idx_flat, z_flat)


def kernel(z, embedding):
    z_flat = z.reshape(-1, _E_DIM)
    zsq = jnp.sum(z_flat ** 2, axis=1, keepdims=True)
    esq = jnp.sum(embedding ** 2, axis=1)
    embt2 = embedding.T * 2.0

    idx_blocks, dmin_blocks = _distance_argmin(z_flat, zsq, embt2,
                                               esq.reshape(1, _N_E))
    encoding_indices = idx_blocks.reshape(z.shape[:-1])
    idx_flat = idx_blocks.reshape(-1)

    table_padded = jnp.pad(embedding, ((0, 0), (0, _ROW_PAD - _E_DIM)))
    z_q_out = _gather_straight_through(table_padded, idx_flat, z_flat)
    z_q_out = z_q_out.reshape(z.shape)

    m = jnp.sum(dmin_blocks) / (8192.0 * _E_DIM)
    loss = _BETA * m + m
    return (z_q_out, loss, encoding_indices)
